# baseline (speedup 1.0000x reference)
import functools

import jax
import jax.numpy as jnp
from jax import lax
from jax.experimental import pallas as pl
from jax.experimental.pallas import tpu as pltpu
from jax.experimental.pallas import tpu_sc as plsc

N_NODES = 10000
N_EDGES = 320000
D = 128

NC = 2
NS = 16
NW = NC * NS
E_PER_W = N_EDGES // NW
CHUNK = 80
NCHUNKS = 128
E_PER_W_P = NCHUNKS * CHUNK
NPAIRS = NCHUNKS // 2
ACC_ROWS = 10240
ROWS_PER_TILE = ACC_ROWS // NS
TRASH_ROW = N_NODES + 16


def _prep_body(f_ref, w1_ref, w2_ref, b1_ref, p_ref, bf_ref):
    wf = jnp.dot(w1_ref[...], w2_ref[...], preferred_element_type=jnp.float32)
    p_ref[...] = jnp.dot(f_ref[...], wf, preferred_element_type=jnp.float32)
    bf_ref[...] = jnp.sum(b1_ref[...] * w2_ref[...], axis=0, keepdims=True)


def _prep(features, W1, b1, W2):
    return pl.pallas_call(
        _prep_body,
        out_shape=(
            jax.ShapeDtypeStruct((N_NODES, D), jnp.float32),
            jax.ShapeDtypeStruct((1, D), jnp.float32),
        ),
    )(features, W1, W2, b1.reshape(-1, 1))


def _combine_body(parts_ref, b_ref, o_ref):
    o_ref[...] = (parts_ref[0, :N_NODES] + parts_ref[1, :N_NODES]
                  + b_ref[...])


def _combine(parts, bias_row):
    return pl.pallas_call(
        _combine_body,
        out_shape=jax.ShapeDtypeStruct((N_NODES, D), jnp.float32),
    )(parts, bias_row)


def _agg_body(x_hbm, src_hbm, dst_hbm, out_hbm,
              src_v, dst_v, rows_a, rows_b, acc_sh,
              gsem_a, gsem_b, ssem_a, ssem_b):
    c = lax.axis_index("c")
    s = lax.axis_index("s")
    wid = s * NC + c

    zero16 = jnp.zeros((16,), jnp.float32)

    def zbody(i, carry):
        for j in range(D // 16):
            rows_a[i, pl.ds(j * 16, 16)] = zero16
        return carry

    lax.fori_loop(0, CHUNK, zbody, 0)
    for k in range(ROWS_PER_TILE // CHUNK):
        pltpu.sync_copy(
            rows_a, acc_sh.at[pl.ds(s * ROWS_PER_TILE + k * CHUNK, CHUNK)])
    plsc.subcore_barrier()

    pltpu.sync_copy(src_hbm.at[wid], src_v)
    pltpu.sync_copy(dst_hbm.at[wid], dst_v)

    def g_start(ci, rows_v, sem):
        return pltpu.async_copy(
            x_hbm.at[src_v.at[pl.ds(ci * CHUNK, CHUNK)]], rows_v, sem)

    def g_wait(ci, rows_v, sem):
        pltpu.make_async_copy(
            x_hbm.at[src_v.at[pl.ds(ci * CHUNK, CHUNK)]], rows_v, sem).wait()

    def s_start(ci, rows_v, sem):
        return pltpu.async_copy(
            rows_v, acc_sh.at[dst_v.at[ci]], sem, add=True)

    def s_wait(ci, rows_v, sem):
        pltpu.make_async_copy(
            rows_v, acc_sh.at[dst_v.at[ci]], sem).wait()

    g_start(0, rows_a, gsem_a)
    g_start(1, rows_b, gsem_b)

    def body(i, carry):
        ca = 2 * i
        cb = 2 * i + 1
        g_wait(ca, rows_a, gsem_a)
        s_start(ca, rows_a, ssem_a)
        g_wait(cb, rows_b, gsem_b)
        s_start(cb, rows_b, ssem_b)
        s_wait(ca, rows_a, ssem_a)
        g_start(ca + 2, rows_a, gsem_a)
        s_wait(cb, rows_b, ssem_b)
        g_start(cb + 2, rows_b, gsem_b)
        return carry

    lax.fori_loop(0, NPAIRS - 1, body, 0)
    ca = NCHUNKS - 2
    cb = NCHUNKS - 1
    g_wait(ca, rows_a, gsem_a)
    s_start(ca, rows_a, ssem_a)
    g_wait(cb, rows_b, gsem_b)
    s_start(cb, rows_b, ssem_b)
    s_wait(ca, rows_a, ssem_a)
    s_wait(cb, rows_b, ssem_b)
    plsc.subcore_barrier()

    pltpu.sync_copy(acc_sh.at[pl.ds(s * ROWS_PER_TILE, ROWS_PER_TILE)],
                    out_hbm.at[c, pl.ds(s * ROWS_PER_TILE, ROWS_PER_TILE)])


@functools.partial(
    pl.kernel,
    out_type=jax.ShapeDtypeStruct((NC, ACC_ROWS, D), jnp.float32),
    mesh=plsc.VectorSubcoreMesh(core_axis_name="c", subcore_axis_name="s"),
    scratch_types=[
        pltpu.VMEM((E_PER_W_P,), jnp.int32),
        pltpu.VMEM((NCHUNKS, CHUNK), jnp.int32),
        pltpu.VMEM((CHUNK, D), jnp.float32),
        pltpu.VMEM((CHUNK, D), jnp.float32),
        pltpu.VMEM_SHARED((ACC_ROWS, D), jnp.float32),
        pltpu.SemaphoreType.DMA,
        pltpu.SemaphoreType.DMA,
        pltpu.SemaphoreType.DMA,
        pltpu.SemaphoreType.DMA,
    ],
)
def _agg(x_hbm, src_hbm, dst_hbm, out_hbm,
         src_v, dst_v, rows_a, rows_b, acc_sh,
         gsem_a, gsem_b, ssem_a, ssem_b):
    _agg_body(x_hbm, src_hbm, dst_hbm, out_hbm,
              src_v, dst_v, rows_a, rows_b, acc_sh,
              gsem_a, gsem_b, ssem_a, ssem_b)


def kernel(features, edge_index, W1, b1, W2, b2):
    ei = edge_index.astype(jnp.int32)
    npad = E_PER_W_P - E_PER_W
    srcw = ei[0].reshape(NW, E_PER_W)
    dstw = ei[1].reshape(NW, E_PER_W)
    src2 = jnp.concatenate(
        [srcw, jnp.zeros((NW, npad), jnp.int32)], axis=1)
    dst3 = jnp.concatenate(
        [dstw, jnp.full((NW, npad), TRASH_ROW, jnp.int32)],
        axis=1).reshape(NW, NCHUNKS, CHUNK)

    p, bf = _prep(features, W1, b1, W2)
    parts1 = _agg(p, src2, dst3)
    g = _combine(parts1, bf)
    parts2 = _agg(g, src2, dst3)
    out = _combine(parts2, b2.reshape(1, D))
    return out


# speedup vs baseline: 1.0777x; 1.0777x over previous
import functools

import jax
import jax.numpy as jnp
from jax import lax
from jax.experimental import pallas as pl
from jax.experimental.pallas import tpu as pltpu
from jax.experimental.pallas import tpu_sc as plsc

N_NODES = 10000
N_EDGES = 320000
D = 128

NC = 2
NS = 16
NW = NC * NS
E_PER_W = N_EDGES // NW
CHUNK = 80
NCHUNKS = 128
E_PER_W_P = NCHUNKS * CHUNK
NPAIRS = NCHUNKS // 2
ACC_ROWS = 10240
ROWS_PER_TILE = ACC_ROWS // NS
TRASH_ROW = N_NODES + 16


def _prep_body(f_ref, w1_ref, w2_ref, b1_ref, p_ref, bf_ref):
    wf = jnp.dot(w1_ref[...], w2_ref[...], preferred_element_type=jnp.float32)
    p_ref[...] = jnp.dot(f_ref[...], wf, preferred_element_type=jnp.float32)
    bf_ref[...] = jnp.sum(b1_ref[...] * w2_ref[...], axis=0, keepdims=True)


def _prep(features, W1, b1, W2):
    return pl.pallas_call(
        _prep_body,
        out_shape=(
            jax.ShapeDtypeStruct((N_NODES, D), jnp.float32),
            jax.ShapeDtypeStruct((1, D), jnp.float32),
        ),
    )(features, W1, W2, b1.reshape(-1, 1))


def _combine_body(parts_ref, b_ref, o_ref):
    o_ref[...] = (parts_ref[0, :N_NODES] + parts_ref[1, :N_NODES]
                  + b_ref[...])


def _combine(parts, bias_row):
    return pl.pallas_call(
        _combine_body,
        out_shape=jax.ShapeDtypeStruct((N_NODES, D), jnp.float32),
    )(parts, bias_row)


def _agg_body(x_hbm, src_hbm, dst_hbm, out_hbm,
              src_v, dst_v, rows_a, rows_b, acc_sh,
              gsem_a, gsem_b, ssem_a, ssem_b):
    c = lax.axis_index("c")
    s = lax.axis_index("s")
    wid = s * NC + c

    zero16 = jnp.zeros((16,), jnp.float32)

    def zbody(i, carry):
        for j in range(D // 16):
            rows_a[i, pl.ds(j * 16, 16)] = zero16
        return carry

    lax.fori_loop(0, CHUNK, zbody, 0)
    for k in range(ROWS_PER_TILE // CHUNK):
        pltpu.sync_copy(
            rows_a, acc_sh.at[pl.ds(s * ROWS_PER_TILE + k * CHUNK, CHUNK)])
    plsc.subcore_barrier()

    pltpu.sync_copy(src_hbm.at[wid], src_v)
    pltpu.sync_copy(dst_hbm.at[wid], dst_v)

    def g_start(ci, rows_v, sem):
        return pltpu.async_copy(
            x_hbm.at[src_v.at[pl.ds(ci * CHUNK, CHUNK)]], rows_v, sem)

    def g_wait(ci, rows_v, sem):
        pltpu.make_async_copy(
            x_hbm.at[src_v.at[pl.ds(ci * CHUNK, CHUNK)]], rows_v, sem).wait()

    def s_sync(ci, rows_v):
        pltpu.sync_copy(rows_v, acc_sh.at[dst_v.at[ci]], add=True)

    g_start(0, rows_a, gsem_a)
    g_start(1, rows_b, gsem_b)

    def body(i, carry):
        ca = 2 * i
        cb = 2 * i + 1
        g_wait(ca, rows_a, gsem_a)
        s_sync(ca, rows_a)
        g_start(ca + 2, rows_a, gsem_a)
        g_wait(cb, rows_b, gsem_b)
        s_sync(cb, rows_b)
        g_start(cb + 2, rows_b, gsem_b)
        return carry

    lax.fori_loop(0, NPAIRS - 1, body, 0)
    ca = NCHUNKS - 2
    cb = NCHUNKS - 1
    g_wait(ca, rows_a, gsem_a)
    s_sync(ca, rows_a)
    g_wait(cb, rows_b, gsem_b)
    s_sync(cb, rows_b)
    plsc.subcore_barrier()

    pltpu.sync_copy(acc_sh.at[pl.ds(s * ROWS_PER_TILE, ROWS_PER_TILE)],
                    out_hbm.at[c, pl.ds(s * ROWS_PER_TILE, ROWS_PER_TILE)])


@functools.partial(
    pl.kernel,
    out_type=jax.ShapeDtypeStruct((NC, ACC_ROWS, D), jnp.float32),
    mesh=plsc.VectorSubcoreMesh(core_axis_name="c", subcore_axis_name="s"),
    scratch_types=[
        pltpu.VMEM((E_PER_W_P,), jnp.int32),
        pltpu.VMEM((NCHUNKS, CHUNK), jnp.int32),
        pltpu.VMEM((CHUNK, D), jnp.float32),
        pltpu.VMEM((CHUNK, D), jnp.float32),
        pltpu.VMEM_SHARED((ACC_ROWS, D), jnp.float32),
        pltpu.SemaphoreType.DMA,
        pltpu.SemaphoreType.DMA,
        pltpu.SemaphoreType.DMA,
        pltpu.SemaphoreType.DMA,
    ],
)
def _agg(x_hbm, src_hbm, dst_hbm, out_hbm,
         src_v, dst_v, rows_a, rows_b, acc_sh,
         gsem_a, gsem_b, ssem_a, ssem_b):
    _agg_body(x_hbm, src_hbm, dst_hbm, out_hbm,
              src_v, dst_v, rows_a, rows_b, acc_sh,
              gsem_a, gsem_b, ssem_a, ssem_b)


def kernel(features, edge_index, W1, b1, W2, b2):
    ei = edge_index.astype(jnp.int32)
    npad = E_PER_W_P - E_PER_W
    srcw = ei[0].reshape(NW, E_PER_W)
    dstw = ei[1].reshape(NW, E_PER_W)
    src2 = jnp.concatenate(
        [srcw, jnp.zeros((NW, npad), jnp.int32)], axis=1)
    dst3 = jnp.concatenate(
        [dstw, jnp.full((NW, npad), TRASH_ROW, jnp.int32)],
        axis=1).reshape(NW, NCHUNKS, CHUNK)

    p, bf = _prep(features, W1, b1, W2)
    parts1 = _agg(p, src2, dst3)
    g = _combine(parts1, bf)
    parts2 = _agg(g, src2, dst3)
    out = _combine(parts2, b2.reshape(1, D))
    return out
